# Initial kernel scaffold; baseline (speedup 1.0000x reference)
import jax
import jax.numpy as jnp
from jax.experimental import pallas as pl


def kernel(batch_x, batch_edge_index, W):
    raise NotImplementedError("write your pallas kernel here")


# speedup vs baseline: 22.7401x; 22.7401x over previous
import functools

import jax
import jax.numpy as jnp
from jax import lax
from jax.experimental import pallas as pl
from jax.experimental.pallas import tpu as pltpu
from jax.experimental.pallas import tpu_sc as plsc

N = 10000
NPAD = 10240
E = 320000
D = 128
H = 2
DH = D // H
NC, NS = 2, 16
NW = NC * NS
K = 125
NCHUNK = (E // NW) // K
ROWS_PER_TILE = NPAD // NS

_mesh = plsc.VectorSubcoreMesh(core_axis_name="c", subcore_axis_name="s")


@functools.partial(
    pl.kernel,
    out_type=jax.ShapeDtypeStruct((NC, NPAD), jnp.float32),
    mesh=_mesh,
    scratch_types=[
        pltpu.VMEM((NCHUNK, K), jnp.int32),
        pltpu.VMEM((128,), jnp.float32),
        pltpu.VMEM((ROWS_PER_TILE,), jnp.float32),
        pltpu.VMEM_SHARED((NPAD,), jnp.float32),
    ],
    compiler_params=pltpu.CompilerParams(use_tc_tiling_on_sc=False),
)
def _deg_kernel(dst_hbm, deg_hbm, idx_v, ones_v, z_v, acc):
    c = lax.axis_index("c")
    s = lax.axis_index("s")
    wid = c * NS + s
    one = jnp.full((16,), 1.0, dtype=jnp.float32)
    zero = jnp.zeros((16,), dtype=jnp.float32)
    for i in range(128 // 16):
        ones_v[pl.ds(i * 16, 16)] = one
    for i in range(ROWS_PER_TILE // 16):
        z_v[pl.ds(i * 16, 16)] = zero
    pltpu.sync_copy(z_v, acc.at[pl.ds(s * ROWS_PER_TILE, ROWS_PER_TILE)])
    plsc.subcore_barrier()
    pltpu.sync_copy(dst_hbm.at[wid], idx_v)

    def chunk(j, carry):
        pltpu.sync_copy(ones_v.at[pl.ds(0, K)], acc.at[idx_v.at[j]], add=True)
        return carry

    lax.fori_loop(0, NCHUNK, chunk, 0)
    plsc.subcore_barrier()
    pltpu.sync_copy(acc.at[pl.ds(s * ROWS_PER_TILE, ROWS_PER_TILE)],
                    deg_hbm.at[c, pl.ds(s * ROWS_PER_TILE, ROWS_PER_TILE)])


def _scale_body(x_ref, w_ref, dp_ref, y_ref, yh_ref, dis_ref):
    deg = dp_ref[0] + dp_ref[1] + 1.0
    dis = lax.rsqrt(deg)
    xw = jnp.dot(x_ref[...], w_ref[...], preferred_element_type=jnp.float32)
    y = xw * dis
    y_ref[...] = y
    yh_ref[0] = y[:, :DH]
    yh_ref[1] = y[:, DH:]
    dis_ref[...] = dis


def _scale_kernel(xpad, W, degp):
    R = 1024
    grid = NPAD // R
    return pl.pallas_call(
        _scale_body,
        grid=(grid,),
        in_specs=[
            pl.BlockSpec((R, D), lambda i: (i, 0)),
            pl.BlockSpec((D, D), lambda i: (0, 0)),
            pl.BlockSpec((NC, R, 1), lambda i: (0, i, 0)),
        ],
        out_specs=[
            pl.BlockSpec((R, D), lambda i: (i, 0)),
            pl.BlockSpec((H, R, DH), lambda i: (0, i, 0)),
            pl.BlockSpec((R, 1), lambda i: (i, 0)),
        ],
        out_shape=[
            jax.ShapeDtypeStruct((NPAD, D), jnp.float32),
            jax.ShapeDtypeStruct((H, NPAD, DH), jnp.float32),
            jax.ShapeDtypeStruct((NPAD, 1), jnp.float32),
        ],
    )(xpad, W, degp)


@functools.partial(
    pl.kernel,
    out_type=jax.ShapeDtypeStruct((H, NC, NPAD, DH), jnp.float32),
    mesh=_mesh,
    scratch_types=[
        pltpu.VMEM((NCHUNK, K), jnp.int32),
        pltpu.VMEM((NCHUNK, K), jnp.int32),
        pltpu.VMEM((K, DH), jnp.float32),
        pltpu.VMEM((128, DH), jnp.float32),
        pltpu.SemaphoreType.DMA,
        pltpu.VMEM_SHARED((NPAD, DH), jnp.float32),
    ],
    compiler_params=pltpu.CompilerParams(use_tc_tiling_on_sc=False),
)
def _agg_kernel(yh_hbm, src_hbm, dst_hbm, out_hbm, sidx, didx, rows, zb, gsem,
                acc):
    c = lax.axis_index("c")
    s = lax.axis_index("s")
    wid = c * NS + s
    zero = jnp.zeros((16,), dtype=jnp.float32)

    def zrow(r, carry):
        for j in range(DH // 16):
            zb[r, pl.ds(j * 16, 16)] = zero
        return carry

    lax.fori_loop(0, 128, zrow, 0)

    pltpu.sync_copy(src_hbm.at[wid], sidx)
    pltpu.sync_copy(dst_hbm.at[wid], didx)

    for h in range(H):
        for k in range(ROWS_PER_TILE // 128):
            pltpu.sync_copy(zb, acc.at[pl.ds(s * ROWS_PER_TILE + k * 128, 128)])
        plsc.subcore_barrier()

        def chunk(j, carry):
            pltpu.async_copy(yh_hbm.at[h].at[sidx.at[j]], rows, gsem).wait()
            pltpu.sync_copy(rows, acc.at[didx.at[j]], add=True)
            return carry

        lax.fori_loop(0, NCHUNK, chunk, 0)
        plsc.subcore_barrier()
        pltpu.sync_copy(acc.at[pl.ds(s * ROWS_PER_TILE, ROWS_PER_TILE)],
                        out_hbm.at[h, c, pl.ds(s * ROWS_PER_TILE,
                                               ROWS_PER_TILE)])
        plsc.subcore_barrier()


def _final_body(q_ref, y_ref, dis_ref, o_ref):
    q = q_ref[...]
    agg = jnp.concatenate([q[0, 0] + q[0, 1], q[1, 0] + q[1, 1]], axis=1)
    o_ref[...] = (agg + y_ref[...]) * dis_ref[...]


def _final_kernel(outp, y, dis):
    R = 1024
    grid = NPAD // R
    return pl.pallas_call(
        _final_body,
        grid=(grid,),
        in_specs=[
            pl.BlockSpec((H, NC, R, DH), lambda i: (0, 0, i, 0)),
            pl.BlockSpec((R, D), lambda i: (i, 0)),
            pl.BlockSpec((R, 1), lambda i: (i, 0)),
        ],
        out_specs=pl.BlockSpec((R, D), lambda i: (i, 0)),
        out_shape=jax.ShapeDtypeStruct((NPAD, D), jnp.float32),
    )(outp, y, dis)


def kernel(batch_x, batch_edge_index, W):
    ei = batch_edge_index.astype(jnp.int32)
    src3 = ei[0].reshape(NW, NCHUNK, K)
    dst3 = ei[1].reshape(NW, NCHUNK, K)
    xpad = jnp.pad(batch_x, ((0, NPAD - N), (0, 0)))

    degp = _deg_kernel(dst3)
    y, yh, dis = _scale_kernel(xpad, W, degp.reshape(NC, NPAD, 1))
    outp = _agg_kernel(yh, src3, dst3)
    out = _final_kernel(outp, y, dis)
    return out[:N]


# speedup vs baseline: 36.5712x; 1.6082x over previous
import functools

import jax
import jax.numpy as jnp
from jax import lax
from jax.experimental import pallas as pl
from jax.experimental.pallas import tpu as pltpu
from jax.experimental.pallas import tpu_sc as plsc

N = 10000
NPAD = 10240
E = 320000
D = 128
H = 2
DH = D // H
NC, NS = 2, 16
NW = NC * NS
K = 125
NCHUNK = (E // NW) // K
ROWS_PER_TILE = NPAD // NS

_mesh = plsc.VectorSubcoreMesh(core_axis_name="c", subcore_axis_name="s")


@functools.partial(
    pl.kernel,
    out_type=jax.ShapeDtypeStruct((NC, NPAD), jnp.float32),
    mesh=_mesh,
    scratch_types=[
        pltpu.VMEM((NCHUNK, K), jnp.int32),
        pltpu.VMEM((128,), jnp.float32),
        pltpu.VMEM((ROWS_PER_TILE,), jnp.float32),
        pltpu.VMEM_SHARED((NPAD,), jnp.float32),
    ],
    compiler_params=pltpu.CompilerParams(use_tc_tiling_on_sc=False),
)
def _deg_kernel(dst_hbm, deg_hbm, idx_v, ones_v, z_v, acc):
    c = lax.axis_index("c")
    s = lax.axis_index("s")
    wid = c * NS + s
    one = jnp.full((16,), 1.0, dtype=jnp.float32)
    zero = jnp.zeros((16,), dtype=jnp.float32)
    for i in range(128 // 16):
        ones_v[pl.ds(i * 16, 16)] = one
    for i in range(ROWS_PER_TILE // 16):
        z_v[pl.ds(i * 16, 16)] = zero
    pltpu.sync_copy(z_v, acc.at[pl.ds(s * ROWS_PER_TILE, ROWS_PER_TILE)])
    plsc.subcore_barrier()
    pltpu.sync_copy(dst_hbm.at[wid], idx_v)

    def chunk(j, carry):
        pltpu.sync_copy(ones_v.at[pl.ds(0, K)], acc.at[idx_v.at[j]], add=True)
        return carry

    lax.fori_loop(0, NCHUNK, chunk, 0)
    plsc.subcore_barrier()
    pltpu.sync_copy(acc.at[pl.ds(s * ROWS_PER_TILE, ROWS_PER_TILE)],
                    deg_hbm.at[c, pl.ds(s * ROWS_PER_TILE, ROWS_PER_TILE)])


def _scale_body(x_ref, w_ref, dp_ref, y_ref, yh_ref, dis_ref):
    deg = dp_ref[0] + dp_ref[1] + 1.0
    dis = lax.rsqrt(deg)
    xw = jnp.dot(x_ref[...], w_ref[...], preferred_element_type=jnp.float32)
    y = xw * dis
    y_ref[...] = y
    yh_ref[0] = y[:, :DH]
    yh_ref[1] = y[:, DH:]
    dis_ref[...] = dis


def _scale_kernel(xpad, W, degp):
    R = 1024
    grid = NPAD // R
    return pl.pallas_call(
        _scale_body,
        grid=(grid,),
        in_specs=[
            pl.BlockSpec((R, D), lambda i: (i, 0)),
            pl.BlockSpec((D, D), lambda i: (0, 0)),
            pl.BlockSpec((NC, R, 1), lambda i: (0, i, 0)),
        ],
        out_specs=[
            pl.BlockSpec((R, D), lambda i: (i, 0)),
            pl.BlockSpec((H, R, DH), lambda i: (0, i, 0)),
            pl.BlockSpec((R, 1), lambda i: (i, 0)),
        ],
        out_shape=[
            jax.ShapeDtypeStruct((NPAD, D), jnp.float32),
            jax.ShapeDtypeStruct((H, NPAD, DH), jnp.float32),
            jax.ShapeDtypeStruct((NPAD, 1), jnp.float32),
        ],
    )(xpad, W, degp)


@functools.partial(
    pl.kernel,
    out_type=jax.ShapeDtypeStruct((H, NC, NPAD, DH), jnp.float32),
    mesh=_mesh,
    scratch_types=[
        pltpu.VMEM((NCHUNK, K), jnp.int32),
        pltpu.VMEM((NCHUNK, K), jnp.int32),
        pltpu.VMEM((4, K, DH), jnp.float32),
        pltpu.VMEM((128, DH), jnp.float32),
        pltpu.SemaphoreType.DMA((4,)),
        pltpu.VMEM_SHARED((NPAD, DH), jnp.float32),
    ],
    compiler_params=pltpu.CompilerParams(use_tc_tiling_on_sc=False),
)
def _agg_kernel(yh_hbm, src_hbm, dst_hbm, out_hbm, sidx, didx, rows, zb, gsem,
                acc):
    c = lax.axis_index("c")
    s = lax.axis_index("s")
    wid = c * NS + s
    zero = jnp.zeros((16,), dtype=jnp.float32)

    def zrow(r, carry):
        for j in range(DH // 16):
            zb[r, pl.ds(j * 16, 16)] = zero
        return carry

    lax.fori_loop(0, 128, zrow, 0)

    pltpu.sync_copy(src_hbm.at[wid], sidx)
    pltpu.sync_copy(dst_hbm.at[wid], didx)

    for h in range(H):
        for k in range(ROWS_PER_TILE // 128):
            pltpu.sync_copy(zb, acc.at[pl.ds(s * ROWS_PER_TILE + k * 128, 128)])
        plsc.subcore_barrier()

        for b in range(4):
            pltpu.async_copy(yh_hbm.at[h].at[sidx.at[b]], rows.at[b],
                             gsem.at[b])

        def chunk4(t, carry):
            for b in range(4):
                j = t * 4 + b
                pltpu.make_async_copy(yh_hbm.at[h].at[sidx.at[j]],
                                      rows.at[b], gsem.at[b]).wait()
                pltpu.sync_copy(rows.at[b], acc.at[didx.at[j]], add=True)

                @pl.when(j + 4 < NCHUNK)
                def _():
                    pltpu.async_copy(yh_hbm.at[h].at[sidx.at[j + 4]],
                                     rows.at[b], gsem.at[b])
            return carry

        lax.fori_loop(0, NCHUNK // 4, chunk4, 0)
        plsc.subcore_barrier()
        pltpu.sync_copy(acc.at[pl.ds(s * ROWS_PER_TILE, ROWS_PER_TILE)],
                        out_hbm.at[h, c, pl.ds(s * ROWS_PER_TILE,
                                               ROWS_PER_TILE)])
        plsc.subcore_barrier()


def _final_body(q_ref, y_ref, dis_ref, o_ref):
    q = q_ref[...]
    agg = jnp.concatenate([q[0, 0] + q[0, 1], q[1, 0] + q[1, 1]], axis=1)
    o_ref[...] = (agg + y_ref[...]) * dis_ref[...]


def _final_kernel(outp, y, dis):
    R = 1024
    grid = NPAD // R
    return pl.pallas_call(
        _final_body,
        grid=(grid,),
        in_specs=[
            pl.BlockSpec((H, NC, R, DH), lambda i: (0, 0, i, 0)),
            pl.BlockSpec((R, D), lambda i: (i, 0)),
            pl.BlockSpec((R, 1), lambda i: (i, 0)),
        ],
        out_specs=pl.BlockSpec((R, D), lambda i: (i, 0)),
        out_shape=jax.ShapeDtypeStruct((NPAD, D), jnp.float32),
    )(outp, y, dis)


def kernel(batch_x, batch_edge_index, W):
    ei = batch_edge_index.astype(jnp.int32)
    src3 = ei[0].reshape(NW, NCHUNK, K)
    dst3 = ei[1].reshape(NW, NCHUNK, K)
    xpad = jnp.pad(batch_x, ((0, NPAD - N), (0, 0)))

    degp = _deg_kernel(dst3)
    y, yh, dis = _scale_kernel(xpad, W, degp.reshape(NC, NPAD, 1))
    outp = _agg_kernel(yh, src3, dst3)
    out = _final_kernel(outp, y, dis)
    return out[:N]


# speedup vs baseline: 45.0529x; 1.2319x over previous
import functools

import jax
import jax.numpy as jnp
from jax import lax
from jax.experimental import pallas as pl
from jax.experimental.pallas import tpu as pltpu
from jax.experimental.pallas import tpu_sc as plsc

N = 10000
NPAD = 10240
E = 320000
D = 128
NC, NS = 2, 16
NW = NC * NS
K = 125
NCHUNK = (E // NW) // K
RPT = N // NS
DEG_RPT = NPAD // NS
NBUF = 4

_mesh = plsc.VectorSubcoreMesh(core_axis_name="c", subcore_axis_name="s")
_sc_params = pltpu.CompilerParams(use_tc_tiling_on_sc=False)


@functools.partial(
    pl.kernel,
    out_type=jax.ShapeDtypeStruct((NC, NPAD), jnp.float32),
    mesh=_mesh,
    scratch_types=[
        pltpu.VMEM((NCHUNK, K), jnp.int32),
        pltpu.VMEM((128,), jnp.float32),
        pltpu.VMEM((DEG_RPT,), jnp.float32),
        pltpu.VMEM_SHARED((NPAD,), jnp.float32),
    ],
    compiler_params=_sc_params,
)
def _deg_kernel(dst_hbm, deg_hbm, idx_v, ones_v, z_v, acc):
    c = lax.axis_index("c")
    s = lax.axis_index("s")
    wid = c * NS + s
    one = jnp.full((16,), 1.0, dtype=jnp.float32)
    zero = jnp.zeros((16,), dtype=jnp.float32)
    for i in range(128 // 16):
        ones_v[pl.ds(i * 16, 16)] = one
    for i in range(DEG_RPT // 16):
        z_v[pl.ds(i * 16, 16)] = zero
    pltpu.sync_copy(z_v, acc.at[pl.ds(s * DEG_RPT, DEG_RPT)])
    plsc.subcore_barrier()
    pltpu.sync_copy(dst_hbm.at[wid], idx_v)

    def chunk(j, carry):
        pltpu.sync_copy(ones_v.at[pl.ds(0, K)], acc.at[idx_v.at[j]], add=True)
        return carry

    lax.fori_loop(0, NCHUNK, chunk, 0)
    plsc.subcore_barrier()
    pltpu.sync_copy(acc.at[pl.ds(s * DEG_RPT, DEG_RPT)],
                    deg_hbm.at[c, pl.ds(s * DEG_RPT, DEG_RPT)])


def _scale_body(x_ref, w_ref, dp_ref, y_ref, ybf_ref, dis_ref):
    deg = dp_ref[0] + dp_ref[1] + 1.0
    dis = lax.rsqrt(deg)
    xw = jnp.dot(x_ref[...], w_ref[...], preferred_element_type=jnp.float32)
    y = xw * dis
    y_ref[...] = y
    ybf_ref[...] = y.astype(jnp.bfloat16)
    dis_ref[...] = dis


def _scale_kernel(x, W, degp):
    R = 1000
    grid = N // R
    return pl.pallas_call(
        _scale_body,
        grid=(grid,),
        in_specs=[
            pl.BlockSpec((R, D), lambda i: (i, 0)),
            pl.BlockSpec((D, D), lambda i: (0, 0)),
            pl.BlockSpec((NC, R, 1), lambda i: (0, i, 0)),
        ],
        out_specs=[
            pl.BlockSpec((R, D), lambda i: (i, 0)),
            pl.BlockSpec((R, D), lambda i: (i, 0)),
            pl.BlockSpec((R, 1), lambda i: (i, 0)),
        ],
        out_shape=[
            jax.ShapeDtypeStruct((N, D), jnp.float32),
            jax.ShapeDtypeStruct((N, D), jnp.bfloat16),
            jax.ShapeDtypeStruct((N, 1), jnp.float32),
        ],
    )(x, W, degp)


@functools.partial(
    pl.kernel,
    out_type=jax.ShapeDtypeStruct((NC, N, D), jnp.bfloat16),
    mesh=_mesh,
    scratch_types=[
        pltpu.VMEM((NCHUNK, K), jnp.int32),
        pltpu.VMEM((NCHUNK, K), jnp.int32),
        pltpu.VMEM((NBUF, K, D), jnp.bfloat16),
        pltpu.VMEM((RPT // 5, D), jnp.bfloat16),
        pltpu.SemaphoreType.DMA((NBUF,)),
        pltpu.VMEM_SHARED((N, D), jnp.bfloat16),
    ],
    compiler_params=_sc_params,
)
def _agg_kernel(ybf_hbm, src_hbm, dst_hbm, out_hbm, sidx, didx, rows, zb,
                gsem, acc):
    c = lax.axis_index("c")
    s = lax.axis_index("s")
    wid = c * NS + s
    zero = jnp.zeros((32,), dtype=jnp.bfloat16)

    def zrow(r, carry):
        for j in range(D // 32):
            zb[r, pl.ds(j * 32, 32)] = zero
        return carry

    lax.fori_loop(0, RPT // 5, zrow, 0)

    pltpu.sync_copy(src_hbm.at[wid], sidx)
    pltpu.sync_copy(dst_hbm.at[wid], didx)

    for k in range(5):
        pltpu.sync_copy(zb, acc.at[pl.ds(s * RPT + k * (RPT // 5), RPT // 5)])
    plsc.subcore_barrier()

    for b in range(NBUF):
        pltpu.async_copy(ybf_hbm.at[sidx.at[b]], rows.at[b], gsem.at[b])

    def chunkn(t, carry):
        for b in range(NBUF):
            j = t * NBUF + b
            pltpu.make_async_copy(ybf_hbm.at[sidx.at[j]], rows.at[b],
                                  gsem.at[b]).wait()
            pltpu.sync_copy(rows.at[b], acc.at[didx.at[j]], add=True)

            @pl.when(j + NBUF < NCHUNK)
            def _():
                pltpu.async_copy(ybf_hbm.at[sidx.at[j + NBUF]], rows.at[b],
                                 gsem.at[b])
        return carry

    lax.fori_loop(0, NCHUNK // NBUF, chunkn, 0)
    plsc.subcore_barrier()
    pltpu.sync_copy(acc.at[pl.ds(s * RPT, RPT)],
                    out_hbm.at[c, pl.ds(s * RPT, RPT)])


def _final_body(q_ref, y_ref, dis_ref, o_ref):
    agg = (q_ref[0].astype(jnp.float32) + q_ref[1].astype(jnp.float32)
           + y_ref[...])
    o_ref[...] = agg * dis_ref[...]


def _final_kernel(outp, y, dis):
    R = 1000
    grid = N // R
    return pl.pallas_call(
        _final_body,
        grid=(grid,),
        in_specs=[
            pl.BlockSpec((NC, R, D), lambda i: (0, i, 0)),
            pl.BlockSpec((R, D), lambda i: (i, 0)),
            pl.BlockSpec((R, 1), lambda i: (i, 0)),
        ],
        out_specs=pl.BlockSpec((R, D), lambda i: (i, 0)),
        out_shape=jax.ShapeDtypeStruct((N, D), jnp.float32),
    )(outp, y, dis)


def kernel(batch_x, batch_edge_index, W):
    ei = batch_edge_index.astype(jnp.int32)
    src3 = ei[0].reshape(NW, NCHUNK, K)
    dst3 = ei[1].reshape(NW, NCHUNK, K)

    degp = _deg_kernel(dst3)
    y, ybf, dis = _scale_kernel(batch_x, W, degp.reshape(NC, NPAD, 1)[:, :N])
    outp = _agg_kernel(ybf, src3, dst3)
    return _final_kernel(outp, y, dis)


# speedup vs baseline: 46.7328x; 1.0373x over previous
import functools

import jax
import jax.numpy as jnp
from jax import lax
from jax.experimental import pallas as pl
from jax.experimental.pallas import tpu as pltpu
from jax.experimental.pallas import tpu_sc as plsc

N = 10000
NPAD = 10240
E = 320000
D = 128
NC, NS = 2, 16
NW = NC * NS
K = 128
NCH = E // K
CPT = NCH // NW
CPT_DEG = NCH // NS
RPT = N // NS
DEG_RPT = NPAD // NS
NBUF = 3

_mesh = plsc.VectorSubcoreMesh(core_axis_name="c", subcore_axis_name="s")
_sc_params = pltpu.CompilerParams(use_tc_tiling_on_sc=False,
                                  needs_layout_passes=False)


def _newton_rsqrt(x):
    i = plsc.bitcast(x, jnp.int32)
    i = jnp.int32(0x5F3759DF) - lax.shift_right_logical(i, 1)
    r = plsc.bitcast(i, jnp.float32)
    for _ in range(3):
        r = r * (1.5 - 0.5 * x * r * r)
    return r


@functools.partial(
    pl.kernel,
    out_type=jax.ShapeDtypeStruct((NPAD,), jnp.float32),
    mesh=_mesh,
    scratch_types=[
        pltpu.VMEM((CPT_DEG + 1, K), jnp.int32),
        pltpu.VMEM((K,), jnp.float32),
        pltpu.VMEM((DEG_RPT,), jnp.float32),
        pltpu.VMEM_SHARED((NPAD,), jnp.float32),
    ],
    compiler_params=_sc_params,
)
def _deg_kernel(dst_hbm, dis_hbm, idx_v, ones_v, z_v, acc):
    c = lax.axis_index("c")
    s = lax.axis_index("s")
    one = jnp.full((16,), 1.0, dtype=jnp.float32)
    zero = jnp.zeros((16,), dtype=jnp.float32)
    for i in range(K // 16):
        ones_v[pl.ds(i * 16, 16)] = one
    for i in range(DEG_RPT // 16):
        z_v[pl.ds(i * 16, 16)] = zero
    pltpu.sync_copy(z_v, acc.at[pl.ds(s * DEG_RPT, DEG_RPT)])
    pltpu.sync_copy(dst_hbm.at[pl.ds(s * CPT_DEG, CPT_DEG)],
                    idx_v.at[pl.ds(0, CPT_DEG)])

    @pl.when(s < NCH - NS * CPT_DEG)
    def _():
        pltpu.sync_copy(dst_hbm.at[pl.ds(NS * CPT_DEG + s, 1)],
                        idx_v.at[pl.ds(CPT_DEG, 1)])

    plsc.subcore_barrier()

    def chunk(j, carry):
        pltpu.sync_copy(ones_v, acc.at[idx_v.at[j]], add=True)
        return carry

    lax.fori_loop(0, CPT_DEG, chunk, 0)

    @pl.when(s < NCH - NS * CPT_DEG)
    def _():
        pltpu.sync_copy(ones_v, acc.at[idx_v.at[CPT_DEG]], add=True)

    plsc.subcore_barrier()
    pltpu.sync_copy(acc.at[pl.ds(s * DEG_RPT, DEG_RPT)], z_v)
    for i in range(DEG_RPT // 16):
        d = z_v[pl.ds(i * 16, 16)]
        z_v[pl.ds(i * 16, 16)] = _newton_rsqrt(d + 1.0)

    @pl.when(c == 0)
    def _():
        pltpu.sync_copy(z_v, dis_hbm.at[pl.ds(s * DEG_RPT, DEG_RPT)])


def _scale_body(x_ref, w_ref, dis_ref, y_ref, ybf_ref):
    dis = dis_ref[...]
    xw = jnp.dot(x_ref[...], w_ref[...], preferred_element_type=jnp.float32)
    y = xw * dis
    y_ref[...] = y
    ybf_ref[...] = y.astype(jnp.bfloat16)


def _scale_kernel(x, W, dis2):
    R = 1000
    grid = N // R
    return pl.pallas_call(
        _scale_body,
        grid=(grid,),
        in_specs=[
            pl.BlockSpec((R, D), lambda i: (i, 0)),
            pl.BlockSpec((D, D), lambda i: (0, 0)),
            pl.BlockSpec((R, 1), lambda i: (i, 0)),
        ],
        out_specs=[
            pl.BlockSpec((R, D), lambda i: (i, 0)),
            pl.BlockSpec((R, D), lambda i: (i, 0)),
        ],
        out_shape=[
            jax.ShapeDtypeStruct((N, D), jnp.float32),
            jax.ShapeDtypeStruct((N, D), jnp.bfloat16),
        ],
    )(x, W, dis2)


@functools.partial(
    pl.kernel,
    out_type=jax.ShapeDtypeStruct((NC, N, D), jnp.bfloat16),
    mesh=_mesh,
    scratch_types=[
        pltpu.VMEM((CPT + 1, K), jnp.int32),
        pltpu.VMEM((CPT + 1, K), jnp.int32),
        pltpu.VMEM((NBUF, K, D), jnp.bfloat16),
        pltpu.VMEM((RPT // 5, D), jnp.bfloat16),
        pltpu.SemaphoreType.DMA((NBUF,)),
        pltpu.VMEM_SHARED((N, D), jnp.bfloat16),
    ],
    compiler_params=_sc_params,
)
def _agg_kernel(ybf_hbm, src_hbm, dst_hbm, out_hbm, sidx, didx, rows, zb,
                gsem, acc):
    c = lax.axis_index("c")
    s = lax.axis_index("s")
    wid = c * NS + s
    zero = jnp.zeros((32,), dtype=jnp.bfloat16)

    def zrow(r, carry):
        for j in range(D // 32):
            zb[r, pl.ds(j * 32, 32)] = zero
        return carry

    lax.fori_loop(0, RPT // 5, zrow, 0)

    pltpu.sync_copy(src_hbm.at[pl.ds(wid * CPT, CPT)], sidx.at[pl.ds(0, CPT)])
    pltpu.sync_copy(dst_hbm.at[pl.ds(wid * CPT, CPT)], didx.at[pl.ds(0, CPT)])

    @pl.when(wid < NCH - NW * CPT)
    def _():
        pltpu.sync_copy(src_hbm.at[pl.ds(NW * CPT + wid, 1)],
                        sidx.at[pl.ds(CPT, 1)])
        pltpu.sync_copy(dst_hbm.at[pl.ds(NW * CPT + wid, 1)],
                        didx.at[pl.ds(CPT, 1)])

    for k in range(5):
        pltpu.sync_copy(zb, acc.at[pl.ds(s * RPT + k * (RPT // 5), RPT // 5)])
    plsc.subcore_barrier()

    for b in range(NBUF):
        pltpu.async_copy(ybf_hbm.at[sidx.at[b]], rows.at[b], gsem.at[b])

    def chunkn(t, carry):
        for b in range(NBUF):
            j = t * NBUF + b
            pltpu.make_async_copy(ybf_hbm.at[sidx.at[j]], rows.at[b],
                                  gsem.at[b]).wait()
            pltpu.sync_copy(rows.at[b], acc.at[didx.at[j]], add=True)

            @pl.when(j + NBUF < CPT)
            def _():
                pltpu.async_copy(ybf_hbm.at[sidx.at[j + NBUF]], rows.at[b],
                                 gsem.at[b])
        return carry

    lax.fori_loop(0, CPT // NBUF, chunkn, 0)

    @pl.when(wid < NCH - NW * CPT)
    def _():
        pltpu.async_copy(ybf_hbm.at[sidx.at[CPT]], rows.at[0],
                         gsem.at[0]).wait()
        pltpu.sync_copy(rows.at[0], acc.at[didx.at[CPT]], add=True)

    plsc.subcore_barrier()
    pltpu.sync_copy(acc.at[pl.ds(s * RPT, RPT)],
                    out_hbm.at[c, pl.ds(s * RPT, RPT)])


def _final_body(q_ref, y_ref, dis_ref, o_ref):
    agg = (q_ref[0].astype(jnp.float32) + q_ref[1].astype(jnp.float32)
           + y_ref[...])
    o_ref[...] = agg * dis_ref[...]


def _final_kernel(outp, y, dis2):
    R = 1000
    grid = N // R
    return pl.pallas_call(
        _final_body,
        grid=(grid,),
        in_specs=[
            pl.BlockSpec((NC, R, D), lambda i: (0, i, 0)),
            pl.BlockSpec((R, D), lambda i: (i, 0)),
            pl.BlockSpec((R, 1), lambda i: (i, 0)),
        ],
        out_specs=pl.BlockSpec((R, D), lambda i: (i, 0)),
        out_shape=jax.ShapeDtypeStruct((N, D), jnp.float32),
    )(outp, y, dis2)


def kernel(batch_x, batch_edge_index, W):
    ei = batch_edge_index.astype(jnp.int32)
    src2 = ei[0].reshape(NCH, K)
    dst2 = ei[1].reshape(NCH, K)

    dis = _deg_kernel(dst2)
    dis2 = dis.reshape(NPAD, 1)
    y, ybf = _scale_kernel(batch_x, W, dis2)
    outp = _agg_kernel(ybf, src2, dst2)
    return _final_kernel(outp, y, dis2)


# speedup vs baseline: 52.8241x; 1.1303x over previous
import functools

import jax
import jax.numpy as jnp
from jax import lax
from jax.experimental import pallas as pl
from jax.experimental.pallas import tpu as pltpu
from jax.experimental.pallas import tpu_sc as plsc

N = 10000
NPAD = 10240
E = 320000
D = 128
NC, NS = 2, 16
NW = NC * NS
K = 128
NCH = E // K
CPT = NCH // NW
CPT_DEG = NCH // NS
RPT = N // NS
DEG_RPT = NPAD // NS
NBUF = 6
GA = 3

_mesh = plsc.VectorSubcoreMesh(core_axis_name="c", subcore_axis_name="s")
_sc_params = pltpu.CompilerParams(use_tc_tiling_on_sc=False,
                                  needs_layout_passes=False)


def _newton_rsqrt(x):
    i = plsc.bitcast(x, jnp.int32)
    i = jnp.int32(0x5F3759DF) - lax.shift_right_logical(i, 1)
    r = plsc.bitcast(i, jnp.float32)
    for _ in range(3):
        r = r * (1.5 - 0.5 * x * r * r)
    return r


@functools.partial(
    pl.kernel,
    out_type=jax.ShapeDtypeStruct((NPAD,), jnp.float32),
    mesh=_mesh,
    scratch_types=[
        pltpu.VMEM((CPT_DEG + 1, 2, K), jnp.int32),
        pltpu.VMEM((K,), jnp.float32),
        pltpu.VMEM((DEG_RPT,), jnp.float32),
        pltpu.SemaphoreType.DMA,
        pltpu.VMEM_SHARED((NPAD,), jnp.float32),
    ],
    compiler_params=_sc_params,
)
def _deg_kernel(edges_hbm, dis_hbm, idx_v, ones_v, z_v, ssem, acc):
    c = lax.axis_index("c")
    s = lax.axis_index("s")
    one = jnp.full((16,), 1.0, dtype=jnp.float32)
    zero = jnp.zeros((16,), dtype=jnp.float32)
    for i in range(K // 16):
        ones_v[pl.ds(i * 16, 16)] = one
    for i in range(DEG_RPT // 16):
        z_v[pl.ds(i * 16, 16)] = zero
    pltpu.sync_copy(z_v, acc.at[pl.ds(s * DEG_RPT, DEG_RPT)])
    pltpu.sync_copy(edges_hbm.at[pl.ds(s * CPT_DEG, CPT_DEG)],
                    idx_v.at[pl.ds(0, CPT_DEG)])

    @pl.when(s < NCH - NS * CPT_DEG)
    def _():
        pltpu.sync_copy(edges_hbm.at[pl.ds(NS * CPT_DEG + s, 1)],
                        idx_v.at[pl.ds(CPT_DEG, 1)])

    plsc.subcore_barrier()

    def chunk4(t, carry):
        for b in range(4):
            pltpu.async_copy(ones_v, acc.at[idx_v.at[t * 4 + b, 1]], ssem,
                             add=True)

        @pl.when(t >= 2)
        def _():
            for b in range(4):
                pltpu.make_async_copy(ones_v, acc.at[idx_v.at[0, 1]],
                                      ssem).wait()
        return carry

    lax.fori_loop(0, CPT_DEG // 4, chunk4, 0)
    for _ in range(8):
        pltpu.make_async_copy(ones_v, acc.at[idx_v.at[0, 1]], ssem).wait()

    @pl.when(s < NCH - NS * CPT_DEG)
    def _():
        pltpu.sync_copy(ones_v, acc.at[idx_v.at[CPT_DEG, 1]], add=True)

    plsc.subcore_barrier()
    pltpu.sync_copy(acc.at[pl.ds(s * DEG_RPT, DEG_RPT)], z_v)
    for i in range(DEG_RPT // 16):
        d = z_v[pl.ds(i * 16, 16)]
        z_v[pl.ds(i * 16, 16)] = _newton_rsqrt(d + 1.0)

    @pl.when(c == 0)
    def _():
        pltpu.sync_copy(z_v, dis_hbm.at[pl.ds(s * DEG_RPT, DEG_RPT)])


def _scale_body(x_ref, w_ref, dis_ref, y_ref, ybf_ref):
    dis = dis_ref[...]
    xw = jnp.dot(x_ref[...], w_ref[...], preferred_element_type=jnp.float32)
    y = xw * dis
    y_ref[...] = y
    ybf_ref[...] = y.astype(jnp.bfloat16)


def _scale_kernel(x, W, dis2):
    R = 1000
    grid = N // R
    return pl.pallas_call(
        _scale_body,
        grid=(grid,),
        in_specs=[
            pl.BlockSpec((R, D), lambda i: (i, 0)),
            pl.BlockSpec((D, D), lambda i: (0, 0)),
            pl.BlockSpec((R, 1), lambda i: (i, 0)),
        ],
        out_specs=[
            pl.BlockSpec((R, D), lambda i: (i, 0)),
            pl.BlockSpec((R, D), lambda i: (i, 0)),
        ],
        out_shape=[
            jax.ShapeDtypeStruct((N, D), jnp.float32),
            jax.ShapeDtypeStruct((N, D), jnp.bfloat16),
        ],
    )(x, W, dis2)


@functools.partial(
    pl.kernel,
    out_type=jax.ShapeDtypeStruct((NC, N, D), jnp.bfloat16),
    mesh=_mesh,
    scratch_types=[
        pltpu.VMEM((CPT + 1, 2, K), jnp.int32),
        pltpu.VMEM((NBUF, K, D), jnp.bfloat16),
        pltpu.VMEM((RPT // 5, D), jnp.bfloat16),
        pltpu.SemaphoreType.DMA((NBUF,)),
        pltpu.SemaphoreType.DMA((NBUF,)),
        pltpu.VMEM_SHARED((N, D), jnp.bfloat16),
    ],
    compiler_params=_sc_params,
)
def _agg_kernel(ybf_hbm, edges_hbm, out_hbm, eidx, rows, zb, gsem, ssem, acc):
    c = lax.axis_index("c")
    s = lax.axis_index("s")
    wid = c * NS + s
    zero = jnp.zeros((32,), dtype=jnp.bfloat16)

    def zrow(r, carry):
        for j in range(D // 32):
            zb[r, pl.ds(j * 32, 32)] = zero
        return carry

    lax.fori_loop(0, RPT // 5, zrow, 0)

    pltpu.sync_copy(edges_hbm.at[pl.ds(wid * CPT, CPT)],
                    eidx.at[pl.ds(0, CPT)])

    @pl.when(wid < NCH - NW * CPT)
    def _():
        pltpu.sync_copy(edges_hbm.at[pl.ds(NW * CPT + wid, 1)],
                        eidx.at[pl.ds(CPT, 1)])

    for k in range(5):
        pltpu.sync_copy(zb, acc.at[pl.ds(s * RPT + k * (RPT // 5), RPT // 5)])
    plsc.subcore_barrier()

    for j in range(GA):
        pltpu.async_copy(ybf_hbm.at[eidx.at[j, 0]], rows.at[j], gsem.at[j])

    def visit(t, carry):
        for b in range(NBUF):
            j = t * NBUF + b
            pltpu.make_async_copy(ybf_hbm.at[eidx.at[j, 0]], rows.at[b],
                                  gsem.at[b]).wait()
            pltpu.async_copy(rows.at[b], acc.at[eidx.at[j, 1]], ssem.at[b],
                             add=True)
            bn = (b + GA) % NBUF

            def fire_next():
                pltpu.make_async_copy(rows.at[bn], acc.at[eidx.at[0, 1]],
                                      ssem.at[bn]).wait()
                pltpu.async_copy(ybf_hbm.at[eidx.at[j + GA, 0]], rows.at[bn],
                                 gsem.at[bn])

            if b >= GA:
                @pl.when(j + GA < CPT)
                def _():
                    fire_next()
            else:
                @pl.when(t > 0)
                def _():
                    fire_next()

                @pl.when(t == 0)
                def _():
                    pltpu.async_copy(ybf_hbm.at[eidx.at[j + GA, 0]],
                                     rows.at[bn], gsem.at[bn])
        return carry

    lax.fori_loop(0, CPT // NBUF, visit, 0)
    for b in range(NBUF):
        pltpu.make_async_copy(rows.at[b], acc.at[eidx.at[0, 1]],
                              ssem.at[b]).wait()

    @pl.when(wid < NCH - NW * CPT)
    def _():
        pltpu.async_copy(ybf_hbm.at[eidx.at[CPT, 0]], rows.at[0],
                         gsem.at[0]).wait()
        pltpu.sync_copy(rows.at[0], acc.at[eidx.at[CPT, 1]], add=True)

    plsc.subcore_barrier()
    pltpu.sync_copy(acc.at[pl.ds(s * RPT, RPT)],
                    out_hbm.at[c, pl.ds(s * RPT, RPT)])


def _final_body(q_ref, y_ref, dis_ref, o_ref):
    agg = (q_ref[0].astype(jnp.float32) + q_ref[1].astype(jnp.float32)
           + y_ref[...])
    o_ref[...] = agg * dis_ref[...]


def _final_kernel(outp, y, dis2):
    R = 1000
    grid = N // R
    return pl.pallas_call(
        _final_body,
        grid=(grid,),
        in_specs=[
            pl.BlockSpec((NC, R, D), lambda i: (0, i, 0)),
            pl.BlockSpec((R, D), lambda i: (i, 0)),
            pl.BlockSpec((R, 1), lambda i: (i, 0)),
        ],
        out_specs=pl.BlockSpec((R, D), lambda i: (i, 0)),
        out_shape=jax.ShapeDtypeStruct((N, D), jnp.float32),
    )(outp, y, dis2)


def kernel(batch_x, batch_edge_index, W):
    ei = batch_edge_index.astype(jnp.int32)
    edges = ei.reshape(2, NCH, K).transpose(1, 0, 2)

    dis = _deg_kernel(edges)
    dis2 = dis.reshape(NPAD, 1)
    y, ybf = _scale_kernel(batch_x, W, dis2)
    outp = _agg_kernel(ybf, edges)
    return _final_kernel(outp, y, dis2)


# speedup vs baseline: 54.5247x; 1.0322x over previous
import functools

import jax
import jax.numpy as jnp
from jax import lax
from jax.experimental import pallas as pl
from jax.experimental.pallas import tpu as pltpu
from jax.experimental.pallas import tpu_sc as plsc

N = 10000
NPAD = 10240
E = 320000
D = 128
NC, NS = 2, 16
NW = NC * NS
K = 128
NCH = E // K
CPT = NCH // NW
CPT_DEG = NCH // NS
RPT = N // NS
DEG_RPT = NPAD // NS
NBUF = 6
GA = 3

_mesh = plsc.VectorSubcoreMesh(core_axis_name="c", subcore_axis_name="s")
_sc_params = pltpu.CompilerParams(use_tc_tiling_on_sc=False,
                                  needs_layout_passes=False)


def _newton_rsqrt(x):
    i = plsc.bitcast(x, jnp.int32)
    i = jnp.int32(0x5F3759DF) - lax.shift_right_logical(i, 1)
    r = plsc.bitcast(i, jnp.float32)
    for _ in range(3):
        r = r * (1.5 - 0.5 * x * r * r)
    return r


@functools.partial(
    pl.kernel,
    out_type=jax.ShapeDtypeStruct((NPAD,), jnp.float32),
    mesh=_mesh,
    scratch_types=[
        pltpu.VMEM((CPT_DEG + 1, 2, K), jnp.int32),
        pltpu.VMEM((K,), jnp.float32),
        pltpu.VMEM((DEG_RPT,), jnp.float32),
        pltpu.SemaphoreType.DMA,
        pltpu.VMEM_SHARED((NPAD,), jnp.float32),
    ],
    compiler_params=_sc_params,
)
def _deg_kernel(edges_hbm, dis_hbm, idx_v, ones_v, z_v, ssem, acc):
    c = lax.axis_index("c")
    s = lax.axis_index("s")
    one = jnp.full((16,), 1.0, dtype=jnp.float32)
    zero = jnp.zeros((16,), dtype=jnp.float32)
    for i in range(K // 16):
        ones_v[pl.ds(i * 16, 16)] = one
    for i in range(DEG_RPT // 16):
        z_v[pl.ds(i * 16, 16)] = zero
    pltpu.sync_copy(z_v, acc.at[pl.ds(s * DEG_RPT, DEG_RPT)])
    pltpu.sync_copy(edges_hbm.at[pl.ds(s * CPT_DEG, CPT_DEG)],
                    idx_v.at[pl.ds(0, CPT_DEG)])

    @pl.when(s < NCH - NS * CPT_DEG)
    def _():
        pltpu.sync_copy(edges_hbm.at[pl.ds(NS * CPT_DEG + s, 1)],
                        idx_v.at[pl.ds(CPT_DEG, 1)])

    plsc.subcore_barrier()

    def chunk4(t, carry):
        for b in range(4):
            pltpu.async_copy(ones_v, acc.at[idx_v.at[t * 4 + b, 1]], ssem,
                             add=True)

        @pl.when(t >= 2)
        def _():
            for b in range(4):
                pltpu.make_async_copy(ones_v, acc.at[idx_v.at[0, 1]],
                                      ssem).wait()
        return carry

    lax.fori_loop(0, CPT_DEG // 4, chunk4, 0)
    for _ in range(8):
        pltpu.make_async_copy(ones_v, acc.at[idx_v.at[0, 1]], ssem).wait()

    @pl.when(s < NCH - NS * CPT_DEG)
    def _():
        pltpu.sync_copy(ones_v, acc.at[idx_v.at[CPT_DEG, 1]], add=True)

    plsc.subcore_barrier()
    pltpu.sync_copy(acc.at[pl.ds(s * DEG_RPT, DEG_RPT)], z_v)
    for i in range(DEG_RPT // 16):
        d = z_v[pl.ds(i * 16, 16)]
        z_v[pl.ds(i * 16, 16)] = _newton_rsqrt(d + 1.0)

    @pl.when(c == 0)
    def _():
        pltpu.sync_copy(z_v, dis_hbm.at[pl.ds(s * DEG_RPT, DEG_RPT)])


def _scale_body(x_ref, w_ref, dis_ref, y_ref, ybf_ref):
    dis = dis_ref[...]
    xw = jnp.dot(x_ref[...], w_ref[...], preferred_element_type=jnp.float32)
    y = xw * dis
    y_ref[...] = y
    ybf_ref[...] = y.astype(jnp.bfloat16)


def _scale_kernel(x, W, dis2):
    R = 1000
    grid = N // R
    return pl.pallas_call(
        _scale_body,
        grid=(grid,),
        in_specs=[
            pl.BlockSpec((R, D), lambda i: (i, 0)),
            pl.BlockSpec((D, D), lambda i: (0, 0)),
            pl.BlockSpec((R, 1), lambda i: (i, 0)),
        ],
        out_specs=[
            pl.BlockSpec((R, D), lambda i: (i, 0)),
            pl.BlockSpec((R, D), lambda i: (i, 0)),
        ],
        out_shape=[
            jax.ShapeDtypeStruct((N, D), jnp.float32),
            jax.ShapeDtypeStruct((N, D), jnp.bfloat16),
        ],
    )(x, W, dis2)


@functools.partial(
    pl.kernel,
    out_type=jax.ShapeDtypeStruct((NC, N, D), jnp.bfloat16),
    mesh=_mesh,
    scratch_types=[
        pltpu.VMEM((CPT + 1, 2, K), jnp.int32),
        pltpu.VMEM((NBUF, K, D), jnp.bfloat16),
        pltpu.VMEM((RPT // 5, D), jnp.bfloat16),
        pltpu.SemaphoreType.DMA((NBUF,)),
        pltpu.VMEM_SHARED((N, D), jnp.bfloat16),
    ],
    compiler_params=_sc_params,
)
def _agg_kernel(ybf_hbm, edges_hbm, out_hbm, eidx, rows, zb, gsem, acc):
    c = lax.axis_index("c")
    s = lax.axis_index("s")
    wid = c * NS + s
    zero = jnp.zeros((32,), dtype=jnp.bfloat16)

    def zrow(r, carry):
        for j in range(D // 32):
            zb[r, pl.ds(j * 32, 32)] = zero
        return carry

    lax.fori_loop(0, RPT // 5, zrow, 0)

    pltpu.sync_copy(edges_hbm.at[pl.ds(wid * CPT, CPT)],
                    eidx.at[pl.ds(0, CPT)])

    @pl.when(wid < NCH - NW * CPT)
    def _():
        pltpu.sync_copy(edges_hbm.at[pl.ds(NW * CPT + wid, 1)],
                        eidx.at[pl.ds(CPT, 1)])

    for k in range(5):
        pltpu.sync_copy(zb, acc.at[pl.ds(s * RPT + k * (RPT // 5), RPT // 5)])
    plsc.subcore_barrier()

    for j in range(GA):
        pltpu.async_copy(ybf_hbm.at[eidx.at[j, 0]], rows.at[j], gsem.at[j])

    def visit(t, carry):
        for b in range(NBUF):
            j = t * NBUF + b
            pltpu.make_async_copy(ybf_hbm.at[eidx.at[j, 0]], rows.at[b],
                                  gsem.at[b]).wait()
            pltpu.sync_copy(rows.at[b], acc.at[eidx.at[j, 1]], add=True)
            bn = (b + GA) % NBUF

            @pl.when(j + GA < CPT)
            def _():
                pltpu.async_copy(ybf_hbm.at[eidx.at[j + GA, 0]], rows.at[bn],
                                 gsem.at[bn])
        return carry

    lax.fori_loop(0, CPT // NBUF, visit, 0)

    @pl.when(wid < NCH - NW * CPT)
    def _():
        pltpu.async_copy(ybf_hbm.at[eidx.at[CPT, 0]], rows.at[0],
                         gsem.at[0]).wait()
        pltpu.sync_copy(rows.at[0], acc.at[eidx.at[CPT, 1]], add=True)

    plsc.subcore_barrier()
    pltpu.sync_copy(acc.at[pl.ds(s * RPT, RPT)],
                    out_hbm.at[c, pl.ds(s * RPT, RPT)])


def _final_body(q_ref, y_ref, dis_ref, o_ref):
    agg = (q_ref[0].astype(jnp.float32) + q_ref[1].astype(jnp.float32)
           + y_ref[...])
    o_ref[...] = agg * dis_ref[...]


def _final_kernel(outp, y, dis2):
    R = 1000
    grid = N // R
    return pl.pallas_call(
        _final_body,
        grid=(grid,),
        in_specs=[
            pl.BlockSpec((NC, R, D), lambda i: (0, i, 0)),
            pl.BlockSpec((R, D), lambda i: (i, 0)),
            pl.BlockSpec((R, 1), lambda i: (i, 0)),
        ],
        out_specs=pl.BlockSpec((R, D), lambda i: (i, 0)),
        out_shape=jax.ShapeDtypeStruct((N, D), jnp.float32),
    )(outp, y, dis2)


def kernel(batch_x, batch_edge_index, W):
    ei = batch_edge_index.astype(jnp.int32)
    edges = ei.reshape(2, NCH, K).transpose(1, 0, 2)

    dis = _deg_kernel(edges)
    dis2 = dis.reshape(NPAD, 1)
    y, ybf = _scale_kernel(batch_x, W, dis2)
    outp = _agg_kernel(ybf, edges)
    return _final_kernel(outp, y, dis2)


# speedup vs baseline: 55.2131x; 1.0126x over previous
import functools

import jax
import jax.numpy as jnp
from jax import lax
from jax.experimental import pallas as pl
from jax.experimental.pallas import tpu as pltpu
from jax.experimental.pallas import tpu_sc as plsc

N = 10000
NPAD = 10240
E = 320000
D = 128
NC, NS = 2, 16
NW = NC * NS
K = 128
NCH = E // K
CPT = NCH // NW
CPT_DEG = NCH // NS
RPT = N // NS
DEG_RPT = NPAD // NS
NBUF = 6
GA = 6

_mesh = plsc.VectorSubcoreMesh(core_axis_name="c", subcore_axis_name="s")
_sc_params = pltpu.CompilerParams(use_tc_tiling_on_sc=False,
                                  needs_layout_passes=False)


def _newton_rsqrt(x):
    i = plsc.bitcast(x, jnp.int32)
    i = jnp.int32(0x5F3759DF) - lax.shift_right_logical(i, 1)
    r = plsc.bitcast(i, jnp.float32)
    for _ in range(3):
        r = r * (1.5 - 0.5 * x * r * r)
    return r


@functools.partial(
    pl.kernel,
    out_type=jax.ShapeDtypeStruct((NPAD,), jnp.float32),
    mesh=_mesh,
    scratch_types=[
        pltpu.VMEM((CPT_DEG + 1, 2, K), jnp.int32),
        pltpu.VMEM((K,), jnp.float32),
        pltpu.VMEM((DEG_RPT,), jnp.float32),
        pltpu.SemaphoreType.DMA,
        pltpu.VMEM_SHARED((NPAD,), jnp.float32),
    ],
    compiler_params=_sc_params,
)
def _deg_kernel(edges_hbm, dis_hbm, idx_v, ones_v, z_v, ssem, acc):
    c = lax.axis_index("c")
    s = lax.axis_index("s")
    one = jnp.full((16,), 1.0, dtype=jnp.float32)
    zero = jnp.zeros((16,), dtype=jnp.float32)
    for i in range(K // 16):
        ones_v[pl.ds(i * 16, 16)] = one
    for i in range(DEG_RPT // 16):
        z_v[pl.ds(i * 16, 16)] = zero
    pltpu.sync_copy(z_v, acc.at[pl.ds(s * DEG_RPT, DEG_RPT)])
    pltpu.sync_copy(edges_hbm.at[pl.ds(s * CPT_DEG, CPT_DEG)],
                    idx_v.at[pl.ds(0, CPT_DEG)])

    @pl.when(s < NCH - NS * CPT_DEG)
    def _():
        pltpu.sync_copy(edges_hbm.at[pl.ds(NS * CPT_DEG + s, 1)],
                        idx_v.at[pl.ds(CPT_DEG, 1)])

    plsc.subcore_barrier()

    def chunk4(t, carry):
        for b in range(4):
            pltpu.async_copy(ones_v, acc.at[idx_v.at[t * 4 + b, 1]], ssem,
                             add=True)

        @pl.when(t >= 2)
        def _():
            for b in range(4):
                pltpu.make_async_copy(ones_v, acc.at[idx_v.at[0, 1]],
                                      ssem).wait()
        return carry

    lax.fori_loop(0, CPT_DEG // 4, chunk4, 0)
    for _ in range(8):
        pltpu.make_async_copy(ones_v, acc.at[idx_v.at[0, 1]], ssem).wait()

    @pl.when(s < NCH - NS * CPT_DEG)
    def _():
        pltpu.sync_copy(ones_v, acc.at[idx_v.at[CPT_DEG, 1]], add=True)

    plsc.subcore_barrier()
    pltpu.sync_copy(acc.at[pl.ds(s * DEG_RPT, DEG_RPT)], z_v)
    for i in range(DEG_RPT // 16):
        d = z_v[pl.ds(i * 16, 16)]
        z_v[pl.ds(i * 16, 16)] = _newton_rsqrt(d + 1.0)

    @pl.when(c == 0)
    def _():
        pltpu.sync_copy(z_v, dis_hbm.at[pl.ds(s * DEG_RPT, DEG_RPT)])


def _scale_body(x_ref, w_ref, dis_ref, y_ref, ybf_ref):
    dis = dis_ref[...]
    xw = jnp.dot(x_ref[...], w_ref[...], preferred_element_type=jnp.float32)
    y = xw * dis
    y_ref[...] = y
    ybf_ref[...] = y.astype(jnp.bfloat16)


def _scale_kernel(x, W, dis2):
    R = 1000
    grid = N // R
    return pl.pallas_call(
        _scale_body,
        grid=(grid,),
        in_specs=[
            pl.BlockSpec((R, D), lambda i: (i, 0)),
            pl.BlockSpec((D, D), lambda i: (0, 0)),
            pl.BlockSpec((R, 1), lambda i: (i, 0)),
        ],
        out_specs=[
            pl.BlockSpec((R, D), lambda i: (i, 0)),
            pl.BlockSpec((R, D), lambda i: (i, 0)),
        ],
        out_shape=[
            jax.ShapeDtypeStruct((N, D), jnp.float32),
            jax.ShapeDtypeStruct((N, D), jnp.bfloat16),
        ],
    )(x, W, dis2)


@functools.partial(
    pl.kernel,
    out_type=jax.ShapeDtypeStruct((NC, N, D), jnp.bfloat16),
    mesh=_mesh,
    scratch_types=[
        pltpu.VMEM((CPT + 1, 2, K), jnp.int32),
        pltpu.VMEM((NBUF, K, D), jnp.bfloat16),
        pltpu.VMEM((RPT // 5, D), jnp.bfloat16),
        pltpu.SemaphoreType.DMA((NBUF,)),
        pltpu.VMEM_SHARED((N, D), jnp.bfloat16),
    ],
    compiler_params=_sc_params,
)
def _agg_kernel(ybf_hbm, edges_hbm, out_hbm, eidx, rows, zb, gsem, acc):
    c = lax.axis_index("c")
    s = lax.axis_index("s")
    wid = c * NS + s
    zero = jnp.zeros((32,), dtype=jnp.bfloat16)

    def zrow(r, carry):
        for j in range(D // 32):
            zb[r, pl.ds(j * 32, 32)] = zero
        return carry

    lax.fori_loop(0, RPT // 5, zrow, 0)

    pltpu.sync_copy(edges_hbm.at[pl.ds(wid * CPT, CPT)],
                    eidx.at[pl.ds(0, CPT)])

    @pl.when(wid < NCH - NW * CPT)
    def _():
        pltpu.sync_copy(edges_hbm.at[pl.ds(NW * CPT + wid, 1)],
                        eidx.at[pl.ds(CPT, 1)])

    for k in range(5):
        pltpu.sync_copy(zb, acc.at[pl.ds(s * RPT + k * (RPT // 5), RPT // 5)])
    plsc.subcore_barrier()

    for j in range(GA):
        pltpu.async_copy(ybf_hbm.at[eidx.at[j, 0]], rows.at[j], gsem.at[j])

    def visit(t, carry):
        for b in range(NBUF):
            j = t * NBUF + b
            pltpu.make_async_copy(ybf_hbm.at[eidx.at[j, 0]], rows.at[b],
                                  gsem.at[b]).wait()
            pltpu.sync_copy(rows.at[b], acc.at[eidx.at[j, 1]], add=True)
            bn = (b + GA) % NBUF

            @pl.when(j + GA < CPT)
            def _():
                pltpu.async_copy(ybf_hbm.at[eidx.at[j + GA, 0]], rows.at[bn],
                                 gsem.at[bn])
        return carry

    lax.fori_loop(0, CPT // NBUF, visit, 0)

    @pl.when(wid < NCH - NW * CPT)
    def _():
        pltpu.async_copy(ybf_hbm.at[eidx.at[CPT, 0]], rows.at[0],
                         gsem.at[0]).wait()
        pltpu.sync_copy(rows.at[0], acc.at[eidx.at[CPT, 1]], add=True)

    plsc.subcore_barrier()
    pltpu.sync_copy(acc.at[pl.ds(s * RPT, RPT)],
                    out_hbm.at[c, pl.ds(s * RPT, RPT)])


def _final_body(q_ref, y_ref, dis_ref, o_ref):
    agg = (q_ref[0].astype(jnp.float32) + q_ref[1].astype(jnp.float32)
           + y_ref[...])
    o_ref[...] = agg * dis_ref[...]


def _final_kernel(outp, y, dis2):
    R = 1000
    grid = N // R
    return pl.pallas_call(
        _final_body,
        grid=(grid,),
        in_specs=[
            pl.BlockSpec((NC, R, D), lambda i: (0, i, 0)),
            pl.BlockSpec((R, D), lambda i: (i, 0)),
            pl.BlockSpec((R, 1), lambda i: (i, 0)),
        ],
        out_specs=pl.BlockSpec((R, D), lambda i: (i, 0)),
        out_shape=jax.ShapeDtypeStruct((N, D), jnp.float32),
    )(outp, y, dis2)


def kernel(batch_x, batch_edge_index, W):
    ei = batch_edge_index.astype(jnp.int32)
    edges = ei.reshape(2, NCH, K).transpose(1, 0, 2)

    dis = _deg_kernel(edges)
    dis2 = dis.reshape(NPAD, 1)
    y, ybf = _scale_kernel(batch_x, W, dis2)
    outp = _agg_kernel(ybf, edges)
    return _final_kernel(outp, y, dis2)


# speedup vs baseline: 55.2534x; 1.0007x over previous
import functools

import jax
import jax.numpy as jnp
from jax import lax
from jax.experimental import pallas as pl
from jax.experimental.pallas import tpu as pltpu
from jax.experimental.pallas import tpu_sc as plsc

N = 10000
NPAD = 10240
E = 320000
D = 128
NC, NS = 2, 16
NW = NC * NS
K = 128
NCH = E // K
CPT = NCH // NW
CPT_DEG = NCH // NS
RPT = N // NS
DEG_RPT = NPAD // NS
NBUF = 6
GA = 6

_mesh = plsc.VectorSubcoreMesh(core_axis_name="c", subcore_axis_name="s")
_sc_params = pltpu.CompilerParams(use_tc_tiling_on_sc=False,
                                  needs_layout_passes=False)


def _newton_rsqrt(x):
    i = plsc.bitcast(x, jnp.int32)
    i = jnp.int32(0x5F3759DF) - lax.shift_right_logical(i, 1)
    r = plsc.bitcast(i, jnp.float32)
    for _ in range(3):
        r = r * (1.5 - 0.5 * x * r * r)
    return r


@functools.partial(
    pl.kernel,
    out_type=jax.ShapeDtypeStruct((NPAD,), jnp.float32),
    mesh=_mesh,
    scratch_types=[
        pltpu.VMEM((CPT_DEG + 1, 2, K), jnp.int32),
        pltpu.VMEM((K,), jnp.float32),
        pltpu.VMEM((DEG_RPT,), jnp.float32),
        pltpu.SemaphoreType.DMA,
        pltpu.VMEM_SHARED((NPAD,), jnp.float32),
    ],
    compiler_params=_sc_params,
)
def _deg_kernel(edges_hbm, dis_hbm, idx_v, ones_v, z_v, ssem, acc):
    c = lax.axis_index("c")
    s = lax.axis_index("s")
    one = jnp.full((16,), 1.0, dtype=jnp.float32)
    zero = jnp.zeros((16,), dtype=jnp.float32)
    for i in range(K // 16):
        ones_v[pl.ds(i * 16, 16)] = one
    for i in range(DEG_RPT // 16):
        z_v[pl.ds(i * 16, 16)] = zero
    pltpu.sync_copy(z_v, acc.at[pl.ds(s * DEG_RPT, DEG_RPT)])
    pltpu.sync_copy(edges_hbm.at[pl.ds(s * CPT_DEG, CPT_DEG)],
                    idx_v.at[pl.ds(0, CPT_DEG)])

    @pl.when(s < NCH - NS * CPT_DEG)
    def _():
        pltpu.sync_copy(edges_hbm.at[pl.ds(NS * CPT_DEG + s, 1)],
                        idx_v.at[pl.ds(CPT_DEG, 1)])

    plsc.subcore_barrier()

    def chunk4(t, carry):
        for b in range(4):
            pltpu.async_copy(ones_v, acc.at[idx_v.at[t * 4 + b, 1]], ssem,
                             add=True)

        @pl.when(t >= 4)
        def _():
            for b in range(4):
                pltpu.make_async_copy(ones_v, acc.at[idx_v.at[0, 1]],
                                      ssem).wait()
        return carry

    lax.fori_loop(0, CPT_DEG // 4, chunk4, 0)
    for _ in range(16):
        pltpu.make_async_copy(ones_v, acc.at[idx_v.at[0, 1]], ssem).wait()

    @pl.when(s < NCH - NS * CPT_DEG)
    def _():
        pltpu.sync_copy(ones_v, acc.at[idx_v.at[CPT_DEG, 1]], add=True)

    plsc.subcore_barrier()
    pltpu.sync_copy(acc.at[pl.ds(s * DEG_RPT, DEG_RPT)], z_v)
    for i in range(DEG_RPT // 16):
        d = z_v[pl.ds(i * 16, 16)]
        z_v[pl.ds(i * 16, 16)] = _newton_rsqrt(d + 1.0)

    @pl.when(c == 0)
    def _():
        pltpu.sync_copy(z_v, dis_hbm.at[pl.ds(s * DEG_RPT, DEG_RPT)])


def _scale_body(x_ref, w_ref, dis_ref, y_ref, ybf_ref):
    dis = dis_ref[...]
    xw = jnp.dot(x_ref[...], w_ref[...], preferred_element_type=jnp.float32)
    y = xw * dis
    y_ref[...] = y
    ybf_ref[...] = y.astype(jnp.bfloat16)


def _scale_kernel(x, W, dis2):
    R = 1000
    grid = N // R
    return pl.pallas_call(
        _scale_body,
        grid=(grid,),
        in_specs=[
            pl.BlockSpec((R, D), lambda i: (i, 0)),
            pl.BlockSpec((D, D), lambda i: (0, 0)),
            pl.BlockSpec((R, 1), lambda i: (i, 0)),
        ],
        out_specs=[
            pl.BlockSpec((R, D), lambda i: (i, 0)),
            pl.BlockSpec((R, D), lambda i: (i, 0)),
        ],
        out_shape=[
            jax.ShapeDtypeStruct((N, D), jnp.float32),
            jax.ShapeDtypeStruct((N, D), jnp.bfloat16),
        ],
    )(x, W, dis2)


@functools.partial(
    pl.kernel,
    out_type=jax.ShapeDtypeStruct((NC, N, D), jnp.bfloat16),
    mesh=_mesh,
    scratch_types=[
        pltpu.VMEM((CPT + 1, 2, K), jnp.int32),
        pltpu.VMEM((NBUF, K, D), jnp.bfloat16),
        pltpu.VMEM((RPT // 5, D), jnp.bfloat16),
        pltpu.SemaphoreType.DMA((NBUF,)),
        pltpu.VMEM_SHARED((N, D), jnp.bfloat16),
    ],
    compiler_params=_sc_params,
)
def _agg_kernel(ybf_hbm, edges_hbm, out_hbm, eidx, rows, zb, gsem, acc):
    c = lax.axis_index("c")
    s = lax.axis_index("s")
    wid = c * NS + s
    zero = jnp.zeros((32,), dtype=jnp.bfloat16)

    def zrow(r, carry):
        for j in range(D // 32):
            zb[r, pl.ds(j * 32, 32)] = zero
        return carry

    lax.fori_loop(0, RPT // 5, zrow, 0)

    pltpu.sync_copy(edges_hbm.at[pl.ds(wid * CPT, CPT)],
                    eidx.at[pl.ds(0, CPT)])

    @pl.when(wid < NCH - NW * CPT)
    def _():
        pltpu.sync_copy(edges_hbm.at[pl.ds(NW * CPT + wid, 1)],
                        eidx.at[pl.ds(CPT, 1)])

    for k in range(5):
        pltpu.sync_copy(zb, acc.at[pl.ds(s * RPT + k * (RPT // 5), RPT // 5)])
    plsc.subcore_barrier()

    for j in range(GA):
        pltpu.async_copy(ybf_hbm.at[eidx.at[j, 0]], rows.at[j], gsem.at[j])

    def visit(t, carry):
        for b in range(NBUF):
            j = t * NBUF + b
            pltpu.make_async_copy(ybf_hbm.at[eidx.at[j, 0]], rows.at[b],
                                  gsem.at[b]).wait()
            pltpu.sync_copy(rows.at[b], acc.at[eidx.at[j, 1]], add=True)
            bn = (b + GA) % NBUF

            @pl.when(j + GA < CPT)
            def _():
                pltpu.async_copy(ybf_hbm.at[eidx.at[j + GA, 0]], rows.at[bn],
                                 gsem.at[bn])
        return carry

    lax.fori_loop(0, CPT // NBUF, visit, 0)

    @pl.when(wid < NCH - NW * CPT)
    def _():
        pltpu.async_copy(ybf_hbm.at[eidx.at[CPT, 0]], rows.at[0],
                         gsem.at[0]).wait()
        pltpu.sync_copy(rows.at[0], acc.at[eidx.at[CPT, 1]], add=True)

    plsc.subcore_barrier()
    pltpu.sync_copy(acc.at[pl.ds(s * RPT, RPT)],
                    out_hbm.at[c, pl.ds(s * RPT, RPT)])


def _final_body(q_ref, y_ref, dis_ref, o_ref):
    agg = (q_ref[0].astype(jnp.float32) + q_ref[1].astype(jnp.float32)
           + y_ref[...])
    o_ref[...] = agg * dis_ref[...]


def _final_kernel(outp, y, dis2):
    R = 1000
    grid = N // R
    return pl.pallas_call(
        _final_body,
        grid=(grid,),
        in_specs=[
            pl.BlockSpec((NC, R, D), lambda i: (0, i, 0)),
            pl.BlockSpec((R, D), lambda i: (i, 0)),
            pl.BlockSpec((R, 1), lambda i: (i, 0)),
        ],
        out_specs=pl.BlockSpec((R, D), lambda i: (i, 0)),
        out_shape=jax.ShapeDtypeStruct((N, D), jnp.float32),
    )(outp, y, dis2)


def kernel(batch_x, batch_edge_index, W):
    ei = batch_edge_index.astype(jnp.int32)
    edges = ei.reshape(2, NCH, K).transpose(1, 0, 2)

    dis = _deg_kernel(edges)
    dis2 = dis.reshape(NPAD, 1)
    y, ybf = _scale_kernel(batch_x, W, dis2)
    outp = _agg_kernel(ybf, edges)
    return _final_kernel(outp, y, dis2)
